# baseline (speedup 1.0000x reference)
import numpy as np
import jax
import jax.numpy as jnp
from jax.experimental import pallas as pl
from jax.experimental.pallas import tpu as pltpu

BB = 64


def _np_onehot(shape, rows, cols):
    m = np.zeros(shape, np.float32)
    m[rows, cols] = 1.0
    return m


def _input_onehot():
    e = np.zeros((3, 64, 528), np.float32)
    xs = np.arange(64)
    for c in range(3):
        e[c, xs, (xs + 1) * 8 + c] = 1.0
    return e.reshape(192, 528)


def _toeplitz(w, cin, cout, wout, wp):
    tks = []
    w = w.astype(jnp.bfloat16)
    for ky in range(3):
        t = jnp.zeros((wp * cin, wout * cout), jnp.bfloat16)
        for kx in range(3):
            g = jnp.broadcast_to(w[ky, kx][:, None, :], (cin, wout, cout))
            g = jnp.pad(g, ((0, 0), (0, 0), (0, wout * cout)))
            g = g.reshape(cin, wout * (wout + 1) * cout)[:, :wout * wout * cout]
            g = (g.reshape(cin, wout, wout * cout).transpose(1, 0, 2)
                 .reshape(wout * cin, wout * cout))
            t = t.at[kx * cin:kx * cin + wout * cin].add(g)
        tks.append(t)
    return jnp.stack(tks)


def _pool_perm(w, c, pad):
    wl_out = (w // 2 + 2 * pad) * c
    x = np.repeat(np.arange(w // 2), c)
    ch = np.tile(np.arange(c), w // 2)
    col = (x + pad) * c + ch
    pe = _np_onehot((w * c, wl_out), (2 * x) * c + ch, col)
    po = _np_onehot((w * c, wl_out), (2 * x + 1) * c + ch, col)
    return pe, po


CH = 8


def _pool_q(h, hp, pad):
    b = np.repeat(np.arange(CH), h // 2)
    y = np.tile(np.arange(h // 2), CH)
    row = b * hp + y + pad
    qe = _np_onehot((CH * hp, CH * h), row, b * h + 2 * y)
    qo = _np_onehot((CH * hp, CH * h), row, b * h + 2 * y + 1)
    return qe, qo


_E = _input_onehot()
_P1E, _P1O = _pool_perm(64, 8, 1)
_P2E, _P2O = _pool_perm(32, 16, 1)
_P3E, _P3O = _pool_perm(16, 32, 0)
_Q1E, _Q1O = _pool_q(64, 40, 1)
_Q2E, _Q2O = _pool_q(32, 24, 1)
_Q3E, _Q3O = _pool_q(16, 8, 0)


def _fused_kernel(x_ref, e_ref, t1_ref, b1_ref, t2_ref, b2_ref,
                  t3_ref, b3_ref, p1e_ref, p1o_ref, q1e_ref, q1o_ref,
                  p2e_ref, p2o_ref, q2e_ref, q2o_ref,
                  p3e_ref, p3o_ref, q3e_ref, q3o_ref,
                  fw1_ref, fb1_ref, wa_ref, ba_ref, wi_ref, bi_ref,
                  wf_ref, bf_ref, logits_ref, concepts_ref):
    n = x_ref.shape[0]
    nch = n // CH
    f32 = jnp.float32
    bf16 = jnp.bfloat16

    def dot(a, b):
        return jnp.dot(a.astype(bf16), b, preferred_element_type=f32)

    xc = jnp.concatenate(
        [x_ref[:, c].reshape(n * 64, 64) for c in range(3)],
        axis=1).astype(bf16)
    a0 = jnp.dot(xc, e_ref[...], preferred_element_type=f32)
    xp = jnp.pad(a0.astype(bf16).reshape(n, 64, 528), ((0, 0), (1, 7), (0, 0)))

    def conv(x3, t_ref, b_ref, h):
        wl_in = x3.shape[2]
        sl = jnp.concatenate(
            [x3[:, ky:ky + h].reshape(n * h, wl_in) for ky in range(3)],
            axis=1).astype(bf16)
        a = jnp.dot(sl, t_ref[...], preferred_element_type=f32)
        return jnp.maximum(a + b_ref[...], 0.0).astype(bf16)

    def pool(a, pe_ref, po_ref, qe_ref, qo_ref, h):
        m = jnp.maximum(dot(a, pe_ref[...]), dot(a, po_ref[...])).astype(bf16)
        m = m.reshape(nch, CH * h, m.shape[1])
        chunks = [jnp.maximum(dot(qe_ref[...], m[i]), dot(qo_ref[...], m[i]))
                  for i in range(nch)]
        hp = qe_ref.shape[0] // CH
        return (jnp.concatenate(chunks, axis=0).astype(bf16)
                .reshape(n, hp, m.shape[2]))

    y = conv(xp, t1_ref, b1_ref, 64)
    y = pool(y, p1e_ref, p1o_ref, q1e_ref, q1o_ref, 64)
    y = conv(y, t2_ref, b2_ref, 32)
    y = pool(y, p2e_ref, p2o_ref, q2e_ref, q2o_ref, 32)
    y = conv(y, t3_ref, b3_ref, 16)
    y = pool(y, p3e_ref, p3o_ref, q3e_ref, q3o_ref, 16)

    def dotf(a, b):
        return jnp.dot(a, b, preferred_element_type=f32)

    feats = jnp.concatenate([y[:, yy, :] for yy in range(8)],
                            axis=1).astype(f32)
    h = dotf(feats, fw1_ref[...])
    h = jnp.maximum(h + fb1_ref[...], 0.0)
    concepts = dotf(h, wa_ref[...]) + ba_ref[...]
    z = dotf(concepts, wi_ref[...]) + bi_ref[...]
    z = jnp.maximum(z, 0.0) + h
    logits_ref[...] = dotf(z, wf_ref[...]) + bf_ref[...]
    concepts_ref[...] = concepts


def kernel(x, conv1_w, conv1_b, conv2_w, conv2_b, conv3_w, conv3_b,
           w1t, b1, wat, ba, wit, bi, wft, bf):
    B = x.shape[0]
    f32 = jnp.float32

    t1 = _toeplitz(conv1_w, 8, 8, 64, 66).reshape(3 * 528, 512)
    t2 = _toeplitz(conv2_w, 8, 16, 32, 34).reshape(3 * 272, 512)
    t3 = _toeplitz(conv3_w, 16, 32, 16, 18).reshape(3 * 288, 512)
    b1r = jnp.tile(conv1_b, (1, 64))
    b2r = jnp.tile(conv2_b, (1, 32))
    b3r = jnp.tile(conv3_b, (1, 16))
    fw1 = w1t

    def _whole(a):
        return pl.BlockSpec(a.shape, lambda i: (0,) * a.ndim)

    b16 = jnp.bfloat16
    args = (jnp.asarray(_E, b16), t1, b1r, t2, b2r,
            t3, b3r,
            jnp.asarray(_P1E, b16), jnp.asarray(_P1O, b16),
            jnp.asarray(_Q1E, b16), jnp.asarray(_Q1O, b16),
            jnp.asarray(_P2E, b16), jnp.asarray(_P2O, b16),
            jnp.asarray(_Q2E, b16), jnp.asarray(_Q2O, b16),
            jnp.asarray(_P3E, b16), jnp.asarray(_P3O, b16),
            jnp.asarray(_Q3E, b16), jnp.asarray(_Q3O, b16),
            fw1, b1, wat, ba, wit, bi, wft, bf)

    logits_pad, concepts_pad = pl.pallas_call(
        _fused_kernel,
        out_shape=(
            jax.ShapeDtypeStruct((B, wft.shape[1]), f32),
            jax.ShapeDtypeStruct((B, wat.shape[1]), f32),
        ),
        grid=(B // BB,),
        in_specs=[pl.BlockSpec((BB, x.shape[1], 64, 64), lambda i: (i, 0, 0, 0))]
                 + [_whole(a) for a in args],
        out_specs=(
            pl.BlockSpec((BB, wft.shape[1]), lambda i: (i, 0)),
            pl.BlockSpec((BB, wat.shape[1]), lambda i: (i, 0)),
        ),
        compiler_params=pltpu.CompilerParams(
            dimension_semantics=("parallel",),
            vmem_limit_bytes=64 * 1024 * 1024,
        ),
    )(x, *args)

    return logits_pad[:, :200], concepts_pad[:, :312]


# speedup vs baseline: 1.0479x; 1.0479x over previous
import numpy as np
import jax
import jax.numpy as jnp
from jax.experimental import pallas as pl
from jax.experimental.pallas import tpu as pltpu

BB = 32


def _np_onehot(shape, rows, cols):
    m = np.zeros(shape, np.float32)
    m[rows, cols] = 1.0
    return m


def _input_onehot():
    e = np.zeros((3, 64, 528), np.float32)
    xs = np.arange(64)
    for c in range(3):
        e[c, xs, (xs + 1) * 8 + c] = 1.0
    return e.reshape(192, 528)


def _toeplitz(w, cin, cout, wout, wp):
    tks = []
    w = w.astype(jnp.bfloat16)
    for ky in range(3):
        t = jnp.zeros((wp * cin, wout * cout), jnp.bfloat16)
        for kx in range(3):
            g = jnp.broadcast_to(w[ky, kx][:, None, :], (cin, wout, cout))
            g = jnp.pad(g, ((0, 0), (0, 0), (0, wout * cout)))
            g = g.reshape(cin, wout * (wout + 1) * cout)[:, :wout * wout * cout]
            g = (g.reshape(cin, wout, wout * cout).transpose(1, 0, 2)
                 .reshape(wout * cin, wout * cout))
            t = t.at[kx * cin:kx * cin + wout * cin].add(g)
        tks.append(t)
    return jnp.stack(tks)


def _pool_perm(w, c, pad):
    wl_out = (w // 2 + 2 * pad) * c
    x = np.repeat(np.arange(w // 2), c)
    ch = np.tile(np.arange(c), w // 2)
    col = (x + pad) * c + ch
    pe = _np_onehot((w * c, wl_out), (2 * x) * c + ch, col)
    po = _np_onehot((w * c, wl_out), (2 * x + 1) * c + ch, col)
    return pe, po


CH = 8


def _pool_q(h, hp, pad):
    b = np.repeat(np.arange(CH), h // 2)
    y = np.tile(np.arange(h // 2), CH)
    row = b * hp + y + pad
    qe = _np_onehot((CH * hp, CH * h), row, b * h + 2 * y)
    qo = _np_onehot((CH * hp, CH * h), row, b * h + 2 * y + 1)
    return qe, qo


_E = _input_onehot()
_P1E, _P1O = _pool_perm(64, 8, 1)
_P2E, _P2O = _pool_perm(32, 16, 1)
_P3E, _P3O = _pool_perm(16, 32, 0)
_Q1E, _Q1O = _pool_q(64, 40, 1)
_Q2E, _Q2O = _pool_q(32, 24, 1)
_Q3E, _Q3O = _pool_q(16, 8, 0)


def _fused_kernel(x_ref, e_ref, t1_ref, b1_ref, t2_ref, b2_ref,
                  t3_ref, b3_ref, p1e_ref, p1o_ref, q1e_ref, q1o_ref,
                  p2e_ref, p2o_ref, q2e_ref, q2o_ref,
                  p3e_ref, p3o_ref, q3e_ref, q3o_ref,
                  fw1_ref, fb1_ref, wa_ref, ba_ref, wi_ref, bi_ref,
                  wf_ref, bf_ref, logits_ref, concepts_ref):
    n = x_ref.shape[0]
    nch = n // CH
    f32 = jnp.float32
    bf16 = jnp.bfloat16

    def dot(a, b):
        return jnp.dot(a.astype(bf16), b, preferred_element_type=f32)

    xc = jnp.concatenate(
        [x_ref[:, c].reshape(n * 64, 64) for c in range(3)],
        axis=1).astype(bf16)
    a0 = jnp.dot(xc, e_ref[...], preferred_element_type=f32)
    xp = jnp.pad(a0.astype(bf16).reshape(n, 64, 528), ((0, 0), (1, 7), (0, 0)))

    def conv(x3, t_ref, b_ref, h):
        wl_in = x3.shape[2]
        sl = jnp.concatenate(
            [x3[:, ky:ky + h].reshape(n * h, wl_in) for ky in range(3)],
            axis=1).astype(bf16)
        a = jnp.dot(sl, t_ref[...], preferred_element_type=f32)
        return jnp.maximum(a + b_ref[...], 0.0).astype(bf16)

    def pool(a, pe_ref, po_ref, qe_ref, qo_ref, h):
        m = jnp.maximum(dot(a, pe_ref[...]), dot(a, po_ref[...])).astype(bf16)
        m = m.reshape(nch, CH * h, m.shape[1])
        chunks = [jnp.maximum(dot(qe_ref[...], m[i]), dot(qo_ref[...], m[i]))
                  for i in range(nch)]
        hp = qe_ref.shape[0] // CH
        return (jnp.concatenate(chunks, axis=0).astype(bf16)
                .reshape(n, hp, m.shape[2]))

    y = conv(xp, t1_ref, b1_ref, 64)
    y = pool(y, p1e_ref, p1o_ref, q1e_ref, q1o_ref, 64)
    y = conv(y, t2_ref, b2_ref, 32)
    y = pool(y, p2e_ref, p2o_ref, q2e_ref, q2o_ref, 32)
    y = conv(y, t3_ref, b3_ref, 16)
    y = pool(y, p3e_ref, p3o_ref, q3e_ref, q3o_ref, 16)

    def dotf(a, b):
        return jnp.dot(a, b, preferred_element_type=f32)

    feats = jnp.concatenate([y[:, yy, :] for yy in range(8)],
                            axis=1).astype(f32)
    h = dotf(feats, fw1_ref[...])
    h = jnp.maximum(h + fb1_ref[...], 0.0)
    concepts = dotf(h, wa_ref[...]) + ba_ref[...]
    z = dotf(concepts, wi_ref[...]) + bi_ref[...]
    z = jnp.maximum(z, 0.0) + h
    logits = dotf(z, wf_ref[...]) + bf_ref[...]
    logits_ref[...] = logits[:, :200]
    concepts_ref[...] = concepts[:, :312]


def kernel(x, conv1_w, conv1_b, conv2_w, conv2_b, conv3_w, conv3_b,
           w1t, b1, wat, ba, wit, bi, wft, bf):
    B = x.shape[0]
    f32 = jnp.float32

    t1 = _toeplitz(conv1_w, 8, 8, 64, 66).reshape(3 * 528, 512)
    t2 = _toeplitz(conv2_w, 8, 16, 32, 34).reshape(3 * 272, 512)
    t3 = _toeplitz(conv3_w, 16, 32, 16, 18).reshape(3 * 288, 512)
    b1r = jnp.tile(conv1_b, (1, 64))
    b2r = jnp.tile(conv2_b, (1, 32))
    b3r = jnp.tile(conv3_b, (1, 16))
    fw1 = w1t

    def _whole(a):
        return pl.BlockSpec(a.shape, lambda i: (0,) * a.ndim)

    b16 = jnp.bfloat16
    args = (jnp.asarray(_E, b16), t1, b1r, t2, b2r,
            t3, b3r,
            jnp.asarray(_P1E, b16), jnp.asarray(_P1O, b16),
            jnp.asarray(_Q1E, b16), jnp.asarray(_Q1O, b16),
            jnp.asarray(_P2E, b16), jnp.asarray(_P2O, b16),
            jnp.asarray(_Q2E, b16), jnp.asarray(_Q2O, b16),
            jnp.asarray(_P3E, b16), jnp.asarray(_P3O, b16),
            jnp.asarray(_Q3E, b16), jnp.asarray(_Q3O, b16),
            fw1, b1, wat, ba, wit, bi, wft, bf)

    logits_pad, concepts_pad = pl.pallas_call(
        _fused_kernel,
        out_shape=(
            jax.ShapeDtypeStruct((B, 200), f32),
            jax.ShapeDtypeStruct((B, 312), f32),
        ),
        grid=(B // BB,),
        in_specs=[pl.BlockSpec((BB, x.shape[1], 64, 64), lambda i: (i, 0, 0, 0))]
                 + [_whole(a) for a in args],
        out_specs=(
            pl.BlockSpec((BB, 200), lambda i: (i, 0)),
            pl.BlockSpec((BB, 312), lambda i: (i, 0)),
        ),
        compiler_params=pltpu.CompilerParams(
            dimension_semantics=("parallel",),
            vmem_limit_bytes=64 * 1024 * 1024,
        ),
    )(x, *args)

    return logits_pad, concepts_pad
